# baseline (speedup 1.0000x reference)
import functools

import jax
import jax.numpy as jnp
from jax import lax
from jax.experimental import pallas as pl
from jax.experimental.pallas import tpu as pltpu
from jax.experimental.pallas import tpu_sc as plsc

NUM_FIELDS = 26
CARD1 = 100001
EMB_DIM = 32
BATCH = 16384

_info = plsc.get_sparse_core_info()
NC, NS = _info.num_cores, _info.num_subcores
NW = NC * NS

TS = 33408
T_BASE = (0, TS, 2 * TS)
T_SIZE = (TS, TS, CARD1 - 2 * TS)
CHK = 4096
NCHK = BATCH // CHK


def _sweep(tT, xT):
    mesh = plsc.VectorSubcoreMesh(core_axis_name="c", subcore_axis_name="s")

    @functools.partial(
        pl.kernel,
        out_type=jax.ShapeDtypeStruct((NUM_FIELDS, EMB_DIM, BATCH), jnp.float32),
        mesh=mesh,
        scratch_types=[
            pltpu.VMEM((T_SIZE[0],), jnp.float32),
            pltpu.VMEM((T_SIZE[1],), jnp.float32),
            pltpu.VMEM((T_SIZE[2],), jnp.float32),
            pltpu.VMEM((2, CHK), jnp.int32),
            pltpu.VMEM((2, CHK), jnp.float32),
            pltpu.SemaphoreType.DMA,
            pltpu.SemaphoreType.DMA,
            pltpu.SemaphoreType.DMA,
            pltpu.SemaphoreType.DMA,
            pltpu.SemaphoreType.DMA,
        ],
        compiler_params=pltpu.CompilerParams(
            use_tc_tiling_on_sc=True, needs_layout_passes=False
        ),
    )
    def k(tT_hbm, xT_hbm, out_hbm, rA, rB, rC, ids_v, o_v,
          semA, semB, semC, sem_i, sem_o):
        w = lax.axis_index("s") * NC + lax.axis_index("c")
        rbufs = (rA, rB, rC)
        rsems = (semA, semB, semC)

        def piece_desc(f, k):
            return pltpu.make_async_copy(
                tT_hbm.at[f].at[w, pl.ds(T_BASE[k], T_SIZE[k])],
                rbufs[k],
                rsems[k],
            )

        def ids_desc(f, c):
            return pltpu.make_async_copy(
                xT_hbm.at[f, pl.ds(c * CHK, CHK)], ids_v.at[c % 2], sem_i
            )

        def out_desc(f, c):
            return pltpu.make_async_copy(
                o_v.at[c % 2], out_hbm.at[f].at[w, pl.ds(c * CHK, CHK)], sem_o
            )

        def gather_pass(kp, c):
            base = T_BASE[kp]

            @pl.loop(0, CHK // 16, unroll=4)
            def _g(i):
                ids16 = ids_v[c % 2, pl.ds(i * 16, 16)]
                if kp == 0:
                    m = ids16 < T_SIZE[0]
                elif kp == 2:
                    m = ids16 >= base
                else:
                    m = jnp.logical_and(ids16 >= base, ids16 < base + TS)
                vals = plsc.load_gather(rbufs[kp], [ids16 - base], mask=m)
                prev = (
                    jnp.zeros_like(vals)
                    if kp == 0
                    else o_v[c % 2, pl.ds(i * 16, 16)]
                )
                o_v[c % 2, pl.ds(i * 16, 16)] = jnp.where(m, vals, prev)

        for kp in range(3):
            piece_desc(0, kp).start()
        ids_desc(0, 0).start()

        @pl.loop(0, NUM_FIELDS)
        def _field(f):
            for kp in range(3):
                piece_desc(f, kp).wait()
            for c in range(NCHK):
                ids_desc(f, c).wait()
                if c + 1 < NCHK:
                    ids_desc(f, c + 1).start()
                else:

                    @pl.when(f + 1 < NUM_FIELDS)
                    def _():
                        ids_desc(f + 1, 0).start()

                if c >= 2:
                    out_desc(f, c - 2).wait()
                for kp in range(3):
                    gather_pass(kp, c)
                    if c == NCHK - 1:

                        @pl.when(f + 1 < NUM_FIELDS)
                        def _(kp=kp):
                            piece_desc(f + 1, kp).start()

                out_desc(f, c).start()

            out_desc(f, NCHK - 2).wait()
            out_desc(f, NCHK - 1).wait()

    return k(tT, xT)


@jax.jit
def kernel(x, tables):
    xT = x.T
    tT = jnp.transpose(tables, (0, 2, 1))
    outT = _sweep(tT, xT)
    return jnp.transpose(outT, (2, 0, 1))
